# baseline (speedup 1.0000x reference)
import functools
import math

import jax
import jax.numpy as jnp
from jax.experimental import pallas as pl


def _mm_mask_kernel(a_ref, b_ref, o_ref):
    kk = pl.program_id(0)

    @pl.when(kk == 0)
    def _():
        o_ref[...] = jnp.zeros_like(o_ref)

    o_ref[...] += jnp.dot(a_ref[...], b_ref[...],
                          preferred_element_type=jnp.float32)

    @pl.when(kk == pl.num_programs(0) - 1)
    def _():
        i = jax.lax.broadcasted_iota(jnp.int32, o_ref.shape, 0)
        j = jax.lax.broadcasted_iota(jnp.int32, o_ref.shape, 1)
        o_ref[...] = jnp.where(i == j, 0.0, o_ref[...])


def _mm_mask(A, B, kchunk):
    M, K = A.shape
    _, Nc = B.shape
    g = K // kchunk
    return pl.pallas_call(
        _mm_mask_kernel,
        grid=(g,),
        in_specs=[pl.BlockSpec((M, kchunk), lambda kk: (0, kk)),
                  pl.BlockSpec((kchunk, Nc), lambda kk: (kk, 0))],
        out_specs=pl.BlockSpec((M, Nc), lambda kk: (0, 0)),
        out_shape=jax.ShapeDtypeStruct((M, Nc), jnp.float32),
    )(A, B)


def _gcn_kernel(t_ref, x_ref, w_ref, b_ref, o_ref, *, relu):
    T = t_ref[...]
    deg = jnp.sum(T, axis=1, keepdims=True) + 2.0
    dinv = jax.lax.rsqrt(deg)
    s = jnp.dot(x_ref[...], w_ref[...],
                preferred_element_type=jnp.float32) * dinv
    out = (jnp.dot(T, s, preferred_element_type=jnp.float32) + 2.0 * s)
    out = out * dinv + b_ref[...]
    if relu:
        out = jnp.maximum(out, 0.0)
    o_ref[...] = out


def _gcn_dense(T, x, W, b, relu):
    n = T.shape[0]
    dout = W.shape[1]
    return pl.pallas_call(
        functools.partial(_gcn_kernel, relu=relu),
        out_shape=jax.ShapeDtypeStruct((n, dout), jnp.float32),
    )(T, x, W, b.reshape(1, -1))


def _mmscale_kernel(x_ref, w_ref, d_ref, o_ref):
    o_ref[...] = jnp.dot(x_ref[...], w_ref[...],
                         preferred_element_type=jnp.float32) * d_ref[...]


def _mmscale(x, W, dinv):
    return pl.pallas_call(
        _mmscale_kernel,
        out_shape=jax.ShapeDtypeStruct((x.shape[0], W.shape[1]), jnp.float32),
    )(x, W, dinv)


def _finish_kernel(a_ref, s_ref, d_ref, b_ref, o_ref):
    o_ref[...] = jnp.maximum(
        (a_ref[...] + 2.0 * s_ref[...]) * d_ref[...] + b_ref[...], 0.0)


def _finish_relu(agg, s, dinv, b):
    return pl.pallas_call(
        _finish_kernel,
        out_shape=jax.ShapeDtypeStruct(agg.shape, jnp.float32),
    )(agg, s, dinv, b.reshape(1, -1))


def _head_kernel(a_ref, s_ref, d_ref, b_ref, g_ref, be_ref, o_ref):
    r = (a_ref[...] + 2.0 * s_ref[...]) * d_ref[...] + b_ref[...]
    n = r.shape[0]
    v = r[:, 0:1]
    mean = jnp.sum(v) / n
    var = jnp.sum((v - mean) ** 2) / n
    vn = g_ref[0, 0] * (v - mean) * jax.lax.rsqrt(var + 1e-5) + be_ref[0, 0]
    r1 = jax.nn.sigmoid(vn)
    col = jax.lax.broadcasted_iota(jnp.int32, r.shape, 1)
    o_ref[...] = jnp.where(col == 0, r1, r)


def _head(agg, s, dinv, b, gamma, beta):
    return pl.pallas_call(
        _head_kernel,
        out_shape=jax.ShapeDtypeStruct(agg.shape, jnp.float32),
    )(agg, s, dinv, b.reshape(1, -1), gamma.reshape(1, 1),
      beta.reshape(1, 1))


def kernel(x, edge_index, dW0, db0, dW1, db1, dW2, db2, dW3, db3,
           pw0, pw1, pw2, uW0, ub0, uW1, ub1, uW2, ub2, bn_gamma, bn_beta):
    n0 = x.shape[0]
    src = edge_index[0]
    dst = edge_index[1]

    deg0 = jnp.zeros((n0,), jnp.float32).at[dst].add(1.0) + 2.0
    dinv0 = jax.lax.rsqrt(deg0)[:, None]

    s0 = _mmscale(x, dW0, dinv0)
    agg0 = jnp.zeros((n0, s0.shape[1]), jnp.float32).at[dst].add(s0[src])
    x1 = _finish_relu(agg0, s0, dinv0, db0)

    k1 = int(math.ceil(0.2 * n0))
    w = pw0 / jnp.linalg.norm(pw0)
    score = jnp.tanh(x1 @ w)
    sv1, perm1 = jax.lax.top_k(score, k1)
    pos = jnp.full((n0,), k1, jnp.int32).at[perm1].set(
        jnp.arange(k1, dtype=jnp.int32))
    n0p = ((n0 + 1023) // 1024) * 1024
    R = jnp.zeros((k1, n0p), jnp.float32).at[pos[dst], src].add(
        1.0, mode='drop')
    R = R.at[jnp.arange(k1), perm1].add(1.0)
    C = jnp.zeros((n0p, k1), jnp.float32).at[dst, pos[src]].add(
        1.0, mode='drop')
    C = C.at[perm1, jnp.arange(k1)].add(1.0)
    T1 = _mm_mask(R, C, 1024)
    xp = x1[perm1] * sv1[:, None]
    x2 = _gcn_dense(T1, xp, dW1, db1, relu=True)

    k2 = int(math.ceil(0.2 * k1))
    w = pw1 / jnp.linalg.norm(pw1)
    sv2, perm2 = jax.lax.top_k(jnp.tanh(x2 @ w), k2)
    A2 = T1[perm2, :].at[jnp.arange(k2), perm2].add(1.0)
    C2 = T1[:, perm2].at[perm2, jnp.arange(k2)].add(1.0)
    T2 = _mm_mask(A2, C2, k1)
    xp2 = x2[perm2] * sv2[:, None]
    x3 = _gcn_dense(T2, xp2, dW2, db2, relu=True)

    k3 = int(math.ceil(0.2 * k2))
    w = pw2 / jnp.linalg.norm(pw2)
    sv3, perm3 = jax.lax.top_k(jnp.tanh(x3 @ w), k3)
    A3 = T2[perm3, :].at[jnp.arange(k3), perm3].add(1.0)
    C3 = T2[:, perm3].at[perm3, jnp.arange(k3)].add(1.0)
    T3 = _mm_mask(A3, C3, k2)
    xp3 = x3[perm3] * sv3[:, None]
    x4 = _gcn_dense(T3, xp3, dW3, db3, relu=True)

    u = jnp.zeros_like(x3).at[perm3].set(x4)
    y = _gcn_dense(T2, x3 + u, uW0, ub0, relu=True)
    u = jnp.zeros_like(x2).at[perm2].set(y)
    y = _gcn_dense(T1, x2 + u, uW1, ub1, relu=True)
    u = jnp.zeros_like(x1).at[perm1].set(y)
    xin = x1 + u

    sF = _mmscale(xin, uW2, dinv0)
    aggF = jnp.zeros((n0, sF.shape[1]), jnp.float32).at[dst].add(sF[src])
    return _head(aggF, sF, dinv0, ub2, bn_gamma, bn_beta)
